# baseline (speedup 1.0000x reference)
import functools

import jax
import jax.numpy as jnp
from jax import lax
from jax.experimental import pallas as pl
from jax.experimental.pallas import tpu as pltpu
from jax.experimental.pallas import tpu_sc as plsc

_B = 4096
_S = 200
_K = 9
_NW = 32
_RPW = _B // _NW
_NG = _RPW // 16
_THETAS = (0.25, 0.5, 0.75)
_TEMP = 0.07
_LOG2E = 1.4426950408889634
_QW = 1.0 / 9.0
_AW = 0.08
_N2 = 2 * _B
_BM = 1024


def _sqrt16(a):
    i = plsc.bitcast(a, jnp.int32)
    y = plsc.bitcast(jnp.int32(0x5F3759DF) - (i >> 1), jnp.float32)
    for _ in range(4):
        y = y * (1.5 - 0.5 * a * y * y)
    return jnp.where(a > 0.0, a * y, 0.0)


def _quantile16(ld, theta):
    lo = jnp.zeros((16,), jnp.int32)
    hi = jnp.zeros((16,), jnp.int32) + _K
    for _ in range(4):
        mid = lo + ((hi - lo) >> 1)
        ym = ld(mid)
        go_left = theta <= ym
        lo = jnp.where(go_left, lo, mid)
        hi = jnp.where(go_left, mid, hi)
    idx = hi
    idx_mod = jnp.where(idx == 0, 1, idx)
    x_left = (idx_mod - 5).astype(jnp.float32)
    y_left = ld(jnp.maximum(idx - 1, 0))
    col_s = jnp.clip(idx - 1, 0, _K - 2)
    s_lo = ld(col_s)
    s_hi = ld(col_s + 1)
    slope = s_hi - s_lo
    xn = x_left + (theta - y_left) / slope
    xn = jnp.where(theta < y_left, 0.0, xn)
    zero_slope = slope == 0.0
    u = jnp.where(zero_slope, x_left, xn)
    v = jnp.where(zero_slope, x_left, 4.0)
    return u, v


@functools.cache
def _sc_kernel():
    mesh = plsc.VectorSubcoreMesh(core_axis_name="c", subcore_axis_name="s")
    return pl.kernel(
        _sc_body,
        mesh=mesh,
        out_type=(
            jax.ShapeDtypeStruct((_NW * 16,), jnp.float32),
            jax.ShapeDtypeStruct((_NW * 32,), jnp.float32),
            jax.ShapeDtypeStruct((_NW * 192,), jnp.float32),
        ),
        scratch_types=[
            pltpu.VMEM((_RPW, _S), jnp.float32),
            pltpu.VMEM((_RPW, _K), jnp.float32),
            pltpu.VMEM((_RPW * _K,), jnp.float32),
            pltpu.VMEM((16,), jnp.float32),
            pltpu.VMEM((32,), jnp.float32),
            pltpu.VMEM((192,), jnp.float32),
        ],
        compiler_params=pltpu.CompilerParams(needs_layout_passes=False),
    )


def _sc_body(scores_hbm, pred_hbm, emd_out, max_out, q_out, scores_v,
             pred_v, cdf_v, emd_v, max_v, q_v):
    wid = lax.axis_index("s") * 2 + lax.axis_index("c")
    base = wid * _RPW
    pltpu.sync_copy(scores_hbm.at[pl.ds(base, _RPW), :], scores_v)
    pltpu.sync_copy(pred_hbm.at[pl.ds(base, _RPW), :], pred_v)
    emd_acc = jnp.zeros((16,), jnp.float32)
    cmax = jnp.zeros((16,), jnp.float32) - 3.0e38
    pmax = jnp.zeros((16,), jnp.float32) - 3.0e38
    qacc = [[jnp.zeros((16,), jnp.float32) for _ in range(4)]
            for _ in range(3)]
    for g in range(_NG):
        rows = lax.iota(jnp.int32, 16) + (g * 16)
        row_s = rows * _S
        row_k = rows * _K

        def body(j, cnts, rows=rows):
            jv = jnp.zeros((16,), jnp.int32) + j
            v = plsc.load_gather(scores_v, [rows, jv])
            return tuple(
                cnts[k] + jnp.where(v <= (k - 4.0), 1.0, 0.0)
                for k in range(_K))

        cnts = lax.fori_loop(
            0, _S, body, tuple(jnp.zeros((16,), jnp.float32)
                               for _ in range(_K)))
        sq = jnp.zeros((16,), jnp.float32)
        for k in range(_K):
            kv = jnp.zeros((16,), jnp.int32) + k
            ck = cnts[k] * (1.0 / _S)
            plsc.store_scatter(cdf_v, [row_k + k], ck)
            pk = plsc.load_gather(pred_v, [rows, kv])
            cmax = jnp.maximum(cmax, ck)
            pmax = jnp.maximum(pmax, pk)
            d = pk - ck
            sq = sq + d * d
        emd_acc = emd_acc + _sqrt16(sq * (1.0 / _K))
        ld_c = lambda col, row_k=row_k: plsc.load_gather(cdf_v, [row_k + col])
        ld_p = lambda col, rows=rows: plsc.load_gather(pred_v, [rows, col])
        for t, theta in enumerate(_THETAS):
            uo, vo = _quantile16(ld_c, theta)
            up, vp = _quantile16(ld_p, theta)
            qacc[t][0] = qacc[t][0] + jnp.abs(up - uo)
            qacc[t][1] = qacc[t][1] + jnp.abs(vp - uo)
            qacc[t][2] = qacc[t][2] + jnp.abs(up - vo)
            qacc[t][3] = qacc[t][3] + jnp.abs(vp - vo)
    emd_v[...] = emd_acc
    max_v[pl.ds(0, 16)] = cmax
    max_v[pl.ds(16, 16)] = pmax
    for t in range(3):
        for s in range(4):
            q_v[pl.ds((t * 4 + s) * 16, 16)] = qacc[t][s]
    pltpu.sync_copy(emd_v, emd_out.at[pl.ds(wid * 16, 16)])
    pltpu.sync_copy(max_v, max_out.at[pl.ds(wid * 32, 32)])
    pltpu.sync_copy(q_v, q_out.at[pl.ds(wid * 192, 192)])


def _tc_ntxent_body(zi_ref, zj_ref, out_ref, zn_ref, zb_ref, zs_ref,
                    accr_ref, accc_ref):
    i = pl.program_id(0)

    @pl.when(i == 0)
    def _():
        for ref, half in ((zi_ref, 0), (zj_ref, 1)):
            z = ref[...]
            nrm = jnp.sqrt(jnp.sum(z * z, axis=1, keepdims=True)) + 1e-12
            zn = z / nrm
            zn_ref[pl.ds(half * _B, _B), :] = zn
            zb_ref[pl.ds(half * _B, _B), :] = zn.astype(jnp.bfloat16)
            zs_ref[pl.ds(half * _B, _B), :] = (
                zn * (_LOG2E / _TEMP)).astype(jnp.bfloat16)
        accr_ref[...] = jnp.zeros_like(accr_ref)
        accc_ref[...] = jnp.zeros_like(accc_ref)

    zi = zn_ref[pl.ds(i * _BM, _BM), :]
    zsi = zs_ref[pl.ds(i * _BM, _BM), :]
    zbi = zb_ref[pl.ds(i * _BM, _BM), :]

    def body(j, _):
        zj = zb_ref[pl.ds(j * _BM, _BM), :]
        s = lax.dot_general(zsi, zj, (((1,), (1,)), ((), ())),
                            preferred_element_type=jnp.float32)
        e = jnp.exp2(s.astype(jnp.bfloat16))
        accr_ref[pl.ds(i * _BM, _BM), :] += jnp.sum(
            e, axis=1, keepdims=True, dtype=jnp.float32)

        @pl.when(j > i)
        def _():
            accc_ref[pl.ds(j, 1), :] += jnp.sum(
                e, axis=0, keepdims=True, dtype=jnp.float32)

        return 0

    lax.fori_loop(i, _N2 // _BM, body, 0)
    selfd = jnp.sum(zsi.astype(jnp.float32) * zbi.astype(jnp.float32),
                    axis=1, keepdims=True)
    selfe = jnp.exp2(selfd.astype(jnp.bfloat16)).astype(jnp.float32)
    colpart = accc_ref[pl.ds(i, 1), :].reshape(_BM, 1)
    lse = jnp.log(accr_ref[pl.ds(i * _BM, _BM), :] + colpart - selfe)
    p = lax.rem(i * _BM + _B, _N2)
    zp = zn_ref[pl.ds(p, _BM), :]
    pos = jnp.sum(zi * zp, axis=1, keepdims=True) * (1.0 / _TEMP)
    out_ref[...] = jnp.full((1, 1, 128), jnp.sum(lse - pos), jnp.float32)


def _tc_ntxent(zi, zj):
    return pl.pallas_call(
        _tc_ntxent_body,
        grid=(_N2 // _BM,),
        in_specs=[pl.BlockSpec((_B, 32), lambda i: (0, 0)),
                  pl.BlockSpec((_B, 32), lambda i: (0, 0))],
        out_specs=pl.BlockSpec((1, 1, 128), lambda i: (i, 0, 0)),
        out_shape=jax.ShapeDtypeStruct((_N2 // _BM, 1, 128), jnp.float32),
        scratch_shapes=[
            pltpu.VMEM((_N2, 32), jnp.float32),
            pltpu.VMEM((_N2, 32), jnp.bfloat16),
            pltpu.VMEM((_N2, 32), jnp.bfloat16),
            pltpu.VMEM((_N2, 1), jnp.float32),
            pltpu.VMEM((_N2 // _BM, _BM), jnp.float32),
        ],
    )(zi, zj)


def kernel(texture_img_f, depth_img_f, original_scores, predicted_cdf):
    emd_parts, max_parts, q_parts = _sc_kernel()(
        original_scores, predicted_cdf)
    lse_parts = _tc_ntxent(texture_img_f, depth_img_f)
    img = jnp.sum(lse_parts[:, 0, 0]) * (1.0 / _N2)
    maxes = max_parts.reshape(_NW, 2, 16)
    ymax_o = jnp.max(maxes[:, 0, :])
    ymax_p = jnp.max(maxes[:, 1, :])
    qsums = q_parts.reshape(_NW, 3, 4, 16).sum(axis=(0, 3))
    qtot = jnp.float32(0.0)
    for t, theta in enumerate(_THETAS):
        bo = theta > ymax_o
        bp = theta > ymax_p
        qtot = qtot + jnp.where(
            bo,
            jnp.where(bp, qsums[t, 3], qsums[t, 2]),
            jnp.where(bp, qsums[t, 1], qsums[t, 0]))
    return (jnp.sum(emd_parts) + qtot * (_QW / 3.0) + img * _AW)


# speedup vs baseline: 1.0161x; 1.0161x over previous
import functools

import jax
import jax.numpy as jnp
from jax import lax
from jax.experimental import pallas as pl
from jax.experimental.pallas import tpu as pltpu
from jax.experimental.pallas import tpu_sc as plsc

_B = 4096
_S = 200
_K = 9
_NW = 32
_RPW = _B // _NW
_NG = _RPW // 16
_THETAS = (0.25, 0.5, 0.75)
_TEMP = 0.07
_LOG2E = 1.4426950408889634
_QW = 1.0 / 9.0
_AW = 0.08
_N2 = 2 * _B
_BM = 1024


def _sqrt16(a):
    i = plsc.bitcast(a, jnp.int32)
    y = plsc.bitcast(jnp.int32(0x5F3759DF) - (i >> 1), jnp.float32)
    for _ in range(4):
        y = y * (1.5 - 0.5 * a * y * y)
    return jnp.where(a > 0.0, a * y, 0.0)


def _quantile16(ld, theta):
    lo = jnp.zeros((16,), jnp.int32)
    hi = jnp.zeros((16,), jnp.int32) + _K
    for _ in range(4):
        mid = lo + ((hi - lo) >> 1)
        ym = ld(mid)
        go_left = theta <= ym
        lo = jnp.where(go_left, lo, mid)
        hi = jnp.where(go_left, mid, hi)
    idx = hi
    idx_mod = jnp.where(idx == 0, 1, idx)
    x_left = (idx_mod - 5).astype(jnp.float32)
    y_left = ld(jnp.maximum(idx - 1, 0))
    col_s = jnp.clip(idx - 1, 0, _K - 2)
    s_lo = ld(col_s)
    s_hi = ld(col_s + 1)
    slope = s_hi - s_lo
    xn = x_left + (theta - y_left) / slope
    xn = jnp.where(theta < y_left, 0.0, xn)
    zero_slope = slope == 0.0
    u = jnp.where(zero_slope, x_left, xn)
    v = jnp.where(zero_slope, x_left, 4.0)
    return u, v


@functools.cache
def _sc_kernel():
    mesh = plsc.VectorSubcoreMesh(core_axis_name="c", subcore_axis_name="s")
    return pl.kernel(
        _sc_body,
        mesh=mesh,
        out_type=(
            jax.ShapeDtypeStruct((_NW * 16,), jnp.float32),
            jax.ShapeDtypeStruct((_NW * 32,), jnp.float32),
            jax.ShapeDtypeStruct((_NW * 192,), jnp.float32),
        ),
        scratch_types=[
            pltpu.VMEM((_RPW, _S), jnp.float32),
            pltpu.VMEM((_RPW, _K), jnp.float32),
            pltpu.VMEM((_RPW * _K,), jnp.float32),
            pltpu.VMEM((16,), jnp.float32),
            pltpu.VMEM((32,), jnp.float32),
            pltpu.VMEM((192,), jnp.float32),
        ],
        compiler_params=pltpu.CompilerParams(needs_layout_passes=False),
    )


def _sc_body(scores_hbm, pred_hbm, emd_out, max_out, q_out, scores_v,
             pred_v, cdf_v, emd_v, max_v, q_v):
    wid = lax.axis_index("s") * 2 + lax.axis_index("c")
    base = wid * _RPW
    pltpu.sync_copy(scores_hbm.at[pl.ds(base, _RPW), :], scores_v)
    pltpu.sync_copy(pred_hbm.at[pl.ds(base, _RPW), :], pred_v)
    emd_acc = jnp.zeros((16,), jnp.float32)
    cmax = jnp.zeros((16,), jnp.float32) - 3.0e38
    pmax = jnp.zeros((16,), jnp.float32) - 3.0e38
    qacc = [[jnp.zeros((16,), jnp.float32) for _ in range(4)]
            for _ in range(3)]
    for g in range(_NG):
        rows = lax.iota(jnp.int32, 16) + (g * 16)
        row_s = rows * _S
        row_k = rows * _K

        def body(j, cnts, rows=rows):
            jv = jnp.zeros((16,), jnp.int32) + j
            v = plsc.load_gather(scores_v, [rows, jv])
            return tuple(
                cnts[k] + jnp.where(v <= (k - 4.0), 1.0, 0.0)
                for k in range(_K))

        cnts = lax.fori_loop(
            0, _S, body, tuple(jnp.zeros((16,), jnp.float32)
                               for _ in range(_K)))
        sq = jnp.zeros((16,), jnp.float32)
        for k in range(_K):
            kv = jnp.zeros((16,), jnp.int32) + k
            ck = cnts[k] * (1.0 / _S)
            plsc.store_scatter(cdf_v, [row_k + k], ck)
            pk = plsc.load_gather(pred_v, [rows, kv])
            cmax = jnp.maximum(cmax, ck)
            pmax = jnp.maximum(pmax, pk)
            d = pk - ck
            sq = sq + d * d
        emd_acc = emd_acc + _sqrt16(sq * (1.0 / _K))
        ld_c = lambda col, row_k=row_k: plsc.load_gather(cdf_v, [row_k + col])
        ld_p = lambda col, rows=rows: plsc.load_gather(pred_v, [rows, col])
        for t, theta in enumerate(_THETAS):
            uo, vo = _quantile16(ld_c, theta)
            up, vp = _quantile16(ld_p, theta)
            qacc[t][0] = qacc[t][0] + jnp.abs(up - uo)
            qacc[t][1] = qacc[t][1] + jnp.abs(vp - uo)
            qacc[t][2] = qacc[t][2] + jnp.abs(up - vo)
            qacc[t][3] = qacc[t][3] + jnp.abs(vp - vo)
    emd_v[...] = emd_acc
    max_v[pl.ds(0, 16)] = cmax
    max_v[pl.ds(16, 16)] = pmax
    for t in range(3):
        for s in range(4):
            q_v[pl.ds((t * 4 + s) * 16, 16)] = qacc[t][s]
    pltpu.sync_copy(emd_v, emd_out.at[pl.ds(wid * 16, 16)])
    pltpu.sync_copy(max_v, max_out.at[pl.ds(wid * 32, 32)])
    pltpu.sync_copy(q_v, q_out.at[pl.ds(wid * 192, 192)])


def _tc_ntxent_body(zi_ref, zj_ref, out_ref, zn_ref, zb_ref, zs_ref,
                    eacc_ref, accc_ref):
    i = pl.program_id(0)

    @pl.when(i == 0)
    def _():
        for ref, half in ((zi_ref, 0), (zj_ref, 1)):
            z = ref[...]
            nrm = jnp.sqrt(jnp.sum(z * z, axis=1, keepdims=True)) + 1e-12
            zn = z / nrm
            zn_ref[pl.ds(half * _B, _B), :] = zn
            zb_ref[pl.ds(half * _B, _B), :] = zn.astype(jnp.bfloat16)
            zs_ref[pl.ds(half * _B, _B), :] = (
                zn * (_LOG2E / _TEMP)).astype(jnp.bfloat16)
        accc_ref[...] = jnp.zeros_like(accc_ref)

    zi = zn_ref[pl.ds(i * _BM, _BM), :]
    zsi = zs_ref[pl.ds(i * _BM, _BM), :]
    zbi = zb_ref[pl.ds(i * _BM, _BM), :]

    def body(j, _):
        zj = zb_ref[pl.ds(j * _BM, _BM), :]
        s = lax.dot_general(zsi, zj, (((1,), (1,)), ((), ())),
                            preferred_element_type=jnp.float32)
        e = jnp.exp2(s)
        eacc_ref[...] += e

        @pl.when(j > i)
        def _():
            accc_ref[pl.ds(j, 1), :] += jnp.sum(
                e, axis=0, keepdims=True, dtype=jnp.float32)

        return 0

    eacc_ref[...] = jnp.zeros_like(eacc_ref)
    lax.fori_loop(i, _N2 // _BM, body, 0)
    rowtot = jnp.sum(eacc_ref[...], axis=1, keepdims=True)
    selfd = jnp.sum(zsi.astype(jnp.float32) * zbi.astype(jnp.float32),
                    axis=1, keepdims=True)
    selfe = jnp.exp2(selfd)
    colpart = accc_ref[pl.ds(i, 1), :].reshape(_BM, 1)
    lse = jnp.log(rowtot + colpart - selfe)
    p = lax.rem(i * _BM + _B, _N2)
    zp = zn_ref[pl.ds(p, _BM), :]
    pos = jnp.sum(zi * zp, axis=1, keepdims=True) * (1.0 / _TEMP)
    out_ref[...] = jnp.full((1, 1, 128), jnp.sum(lse - pos), jnp.float32)


def _tc_ntxent(zi, zj):
    return pl.pallas_call(
        _tc_ntxent_body,
        grid=(_N2 // _BM,),
        in_specs=[pl.BlockSpec((_B, 32), lambda i: (0, 0)),
                  pl.BlockSpec((_B, 32), lambda i: (0, 0))],
        out_specs=pl.BlockSpec((1, 1, 128), lambda i: (i, 0, 0)),
        out_shape=jax.ShapeDtypeStruct((_N2 // _BM, 1, 128), jnp.float32),
        scratch_shapes=[
            pltpu.VMEM((_N2, 32), jnp.float32),
            pltpu.VMEM((_N2, 32), jnp.bfloat16),
            pltpu.VMEM((_N2, 32), jnp.bfloat16),
            pltpu.VMEM((_BM, _BM), jnp.float32),
            pltpu.VMEM((_N2 // _BM, _BM), jnp.float32),
        ],
    )(zi, zj)


def kernel(texture_img_f, depth_img_f, original_scores, predicted_cdf):
    emd_parts, max_parts, q_parts = _sc_kernel()(
        original_scores, predicted_cdf)
    lse_parts = _tc_ntxent(texture_img_f, depth_img_f)
    img = jnp.sum(lse_parts[:, 0, 0]) * (1.0 / _N2)
    maxes = max_parts.reshape(_NW, 2, 16)
    ymax_o = jnp.max(maxes[:, 0, :])
    ymax_p = jnp.max(maxes[:, 1, :])
    qsums = q_parts.reshape(_NW, 3, 4, 16).sum(axis=(0, 3))
    qtot = jnp.float32(0.0)
    for t, theta in enumerate(_THETAS):
        bo = theta > ymax_o
        bp = theta > ymax_p
        qtot = qtot + jnp.where(
            bo,
            jnp.where(bp, qsums[t, 3], qsums[t, 2]),
            jnp.where(bp, qsums[t, 1], qsums[t, 0]))
    return (jnp.sum(emd_parts) + qtot * (_QW / 3.0) + img * _AW)


# speedup vs baseline: 1.0238x; 1.0075x over previous
import functools

import jax
import jax.numpy as jnp
from jax import lax
from jax.experimental import pallas as pl
from jax.experimental.pallas import tpu as pltpu
from jax.experimental.pallas import tpu_sc as plsc

_B = 4096
_S = 200
_K = 9
_NW = 32
_RPW = _B // _NW
_NG = _RPW // 16
_THETAS = (0.25, 0.5, 0.75)
_TEMP = 0.07
_LOG2E = 1.4426950408889634
_QW = 1.0 / 9.0
_AW = 0.08
_N2 = 2 * _B
_BM = 1024


def _sqrt16(a):
    i = plsc.bitcast(a, jnp.int32)
    y = plsc.bitcast(jnp.int32(0x5F3759DF) - (i >> 1), jnp.float32)
    for _ in range(4):
        y = y * (1.5 - 0.5 * a * y * y)
    return jnp.where(a > 0.0, a * y, 0.0)


def _quantile16(ld, theta):
    lo = jnp.zeros((16,), jnp.int32)
    hi = jnp.zeros((16,), jnp.int32) + _K
    for _ in range(4):
        mid = lo + ((hi - lo) >> 1)
        ym = ld(mid)
        go_left = theta <= ym
        lo = jnp.where(go_left, lo, mid)
        hi = jnp.where(go_left, mid, hi)
    idx = hi
    idx_mod = jnp.where(idx == 0, 1, idx)
    x_left = (idx_mod - 5).astype(jnp.float32)
    y_left = ld(jnp.maximum(idx - 1, 0))
    col_s = jnp.clip(idx - 1, 0, _K - 2)
    s_lo = ld(col_s)
    s_hi = ld(col_s + 1)
    slope = s_hi - s_lo
    xn = x_left + (theta - y_left) / slope
    xn = jnp.where(theta < y_left, 0.0, xn)
    zero_slope = slope == 0.0
    u = jnp.where(zero_slope, x_left, xn)
    v = jnp.where(zero_slope, x_left, 4.0)
    return u, v


@functools.cache
def _sc_kernel():
    mesh = plsc.VectorSubcoreMesh(core_axis_name="c", subcore_axis_name="s")
    return pl.kernel(
        _sc_body,
        mesh=mesh,
        out_type=(
            jax.ShapeDtypeStruct((_NW * 16,), jnp.float32),
            jax.ShapeDtypeStruct((_NW * 32,), jnp.float32),
            jax.ShapeDtypeStruct((_NW * 192,), jnp.float32),
        ),
        scratch_types=[
            pltpu.VMEM((_RPW, _S), jnp.float32),
            pltpu.VMEM((_RPW, _K), jnp.float32),
            pltpu.VMEM((_RPW * _K,), jnp.float32),
            pltpu.VMEM((16,), jnp.float32),
            pltpu.VMEM((32,), jnp.float32),
            pltpu.VMEM((192,), jnp.float32),
        ],
        compiler_params=pltpu.CompilerParams(needs_layout_passes=False),
    )


def _sc_body(scores_hbm, pred_hbm, emd_out, max_out, q_out, scores_v,
             pred_v, cdf_v, emd_v, max_v, q_v):
    wid = lax.axis_index("s") * 2 + lax.axis_index("c")
    base = wid * _RPW
    pltpu.sync_copy(scores_hbm.at[pl.ds(base, _RPW), :], scores_v)
    pltpu.sync_copy(pred_hbm.at[pl.ds(base, _RPW), :], pred_v)
    emd_acc = jnp.zeros((16,), jnp.float32)
    cmax = jnp.zeros((16,), jnp.float32) - 3.0e38
    pmax = jnp.zeros((16,), jnp.float32) - 3.0e38
    qacc = [[jnp.zeros((16,), jnp.float32) for _ in range(4)]
            for _ in range(3)]
    for g in range(_NG):
        rows = lax.iota(jnp.int32, 16) + (g * 16)
        row_s = rows * _S
        row_k = rows * _K

        def body(j, cnts, rows=rows):
            jv = jnp.zeros((16,), jnp.int32) + j
            v = plsc.load_gather(scores_v, [rows, jv])
            return tuple(
                cnts[k] + jnp.where(v <= (k - 4.0), 1.0, 0.0)
                for k in range(_K))

        cnts = lax.fori_loop(
            0, _S, body, tuple(jnp.zeros((16,), jnp.float32)
                               for _ in range(_K)))
        sq = jnp.zeros((16,), jnp.float32)
        for k in range(_K):
            kv = jnp.zeros((16,), jnp.int32) + k
            ck = cnts[k] * (1.0 / _S)
            plsc.store_scatter(cdf_v, [row_k + k], ck)
            pk = plsc.load_gather(pred_v, [rows, kv])
            cmax = jnp.maximum(cmax, ck)
            pmax = jnp.maximum(pmax, pk)
            d = pk - ck
            sq = sq + d * d
        emd_acc = emd_acc + _sqrt16(sq * (1.0 / _K))
        ld_c = lambda col, row_k=row_k: plsc.load_gather(cdf_v, [row_k + col])
        ld_p = lambda col, rows=rows: plsc.load_gather(pred_v, [rows, col])
        for t, theta in enumerate(_THETAS):
            uo, vo = _quantile16(ld_c, theta)
            up, vp = _quantile16(ld_p, theta)
            qacc[t][0] = qacc[t][0] + jnp.abs(up - uo)
            qacc[t][1] = qacc[t][1] + jnp.abs(vp - uo)
            qacc[t][2] = qacc[t][2] + jnp.abs(up - vo)
            qacc[t][3] = qacc[t][3] + jnp.abs(vp - vo)
    emd_v[...] = emd_acc
    max_v[pl.ds(0, 16)] = cmax
    max_v[pl.ds(16, 16)] = pmax
    for t in range(3):
        for s in range(4):
            q_v[pl.ds((t * 4 + s) * 16, 16)] = qacc[t][s]
    pltpu.sync_copy(emd_v, emd_out.at[pl.ds(wid * 16, 16)])
    pltpu.sync_copy(max_v, max_out.at[pl.ds(wid * 32, 32)])
    pltpu.sync_copy(q_v, q_out.at[pl.ds(wid * 192, 192)])


def _tc_ntxent_body(zi_ref, zj_ref, out_ref, zn_ref, zb_ref, zs_ref,
                    accr_ref, accc_ref):
    i = pl.program_id(0)

    @pl.when(i == 0)
    def _():
        for ref, half in ((zi_ref, 0), (zj_ref, 1)):
            z = ref[...]
            nrm = jnp.sqrt(jnp.sum(z * z, axis=1, keepdims=True)) + 1e-12
            zn = z / nrm
            zn_ref[pl.ds(half * _B, _B), :] = zn
            zb_ref[pl.ds(half * _B, _B), :] = zn.astype(jnp.bfloat16)
            zs_ref[pl.ds(half * _B, _B), :] = (
                zn * (_LOG2E / _TEMP)).astype(jnp.bfloat16)
        accr_ref[...] = jnp.zeros_like(accr_ref)
        accc_ref[...] = jnp.zeros_like(accc_ref)

    zi = zn_ref[pl.ds(i * _BM, _BM), :]
    zsi = zs_ref[pl.ds(i * _BM, _BM), :]
    zbi = zb_ref[pl.ds(i * _BM, _BM), :]

    def body(j, _):
        zj = zb_ref[pl.ds(j * _BM, _BM), :]
        s = lax.dot_general(zsi, zj, (((1,), (1,)), ((), ())),
                            preferred_element_type=jnp.float32)
        e = jnp.exp2(s)
        accr_ref[pl.ds(i * _BM, _BM), :] += jnp.sum(e, axis=1, keepdims=True)

        @pl.when(j > i)
        def _():
            accc_ref[pl.ds(j, 1), :] += jnp.sum(e, axis=0, keepdims=True)

        return 0

    lax.fori_loop(i, _N2 // _BM, body, 0)
    selfd = jnp.sum(zsi.astype(jnp.float32) * zbi.astype(jnp.float32),
                    axis=1, keepdims=True)
    selfe = jnp.exp2(selfd)
    colpart = accc_ref[pl.ds(i, 1), :].reshape(_BM, 1)
    lse = jnp.log(accr_ref[pl.ds(i * _BM, _BM), :] + colpart - selfe)
    p = lax.rem(i * _BM + _B, _N2)
    zp = zn_ref[pl.ds(p, _BM), :]
    pos = jnp.sum(zi * zp, axis=1, keepdims=True) * (1.0 / _TEMP)
    out_ref[...] = jnp.full((1, 1, 128), jnp.sum(lse - pos), jnp.float32)


def _tc_ntxent(zi, zj):
    return pl.pallas_call(
        _tc_ntxent_body,
        grid=(_N2 // _BM,),
        in_specs=[pl.BlockSpec((_B, 32), lambda i: (0, 0)),
                  pl.BlockSpec((_B, 32), lambda i: (0, 0))],
        out_specs=pl.BlockSpec((1, 1, 128), lambda i: (i, 0, 0)),
        out_shape=jax.ShapeDtypeStruct((_N2 // _BM, 1, 128), jnp.float32),
        scratch_shapes=[
            pltpu.VMEM((_N2, 32), jnp.float32),
            pltpu.VMEM((_N2, 32), jnp.bfloat16),
            pltpu.VMEM((_N2, 32), jnp.bfloat16),
            pltpu.VMEM((_N2, 1), jnp.float32),
            pltpu.VMEM((_N2 // _BM, _BM), jnp.float32),
        ],
    )(zi, zj)


def kernel(texture_img_f, depth_img_f, original_scores, predicted_cdf):
    emd_parts, max_parts, q_parts = _sc_kernel()(
        original_scores, predicted_cdf)
    lse_parts = _tc_ntxent(texture_img_f, depth_img_f)
    img = jnp.sum(lse_parts[:, 0, 0]) * (1.0 / _N2)
    maxes = max_parts.reshape(_NW, 2, 16)
    ymax_o = jnp.max(maxes[:, 0, :])
    ymax_p = jnp.max(maxes[:, 1, :])
    qsums = q_parts.reshape(_NW, 3, 4, 16).sum(axis=(0, 3))
    qtot = jnp.float32(0.0)
    for t, theta in enumerate(_THETAS):
        bo = theta > ymax_o
        bp = theta > ymax_p
        qtot = qtot + jnp.where(
            bo,
            jnp.where(bp, qsums[t, 3], qsums[t, 2]),
            jnp.where(bp, qsums[t, 1], qsums[t, 0]))
    return (jnp.sum(emd_parts) + qtot * (_QW / 3.0) + img * _AW)
